# Initial kernel scaffold; baseline (speedup 1.0000x reference)
import jax
import jax.numpy as jnp
from jax.experimental import pallas as pl


def kernel(x, edge_index, W1, b1, W2, b2, Wr):
    raise NotImplementedError("write your pallas kernel here")


# speedup vs baseline: 13.8934x; 13.8934x over previous
import functools

import jax
import jax.numpy as jnp
from jax import lax
from jax.experimental import pallas as pl
from jax.experimental.pallas import tpu as pltpu
from jax.experimental.pallas import tpu_sc as plsc

N_NODES = 10000
N_PAD = 10240
E_EDGES = 320000
NC = 2
NS = 16
NW = NC * NS
EPW = E_EDGES // NW
DEG_W = 16
R_BLK = 1000
F32 = jnp.float32


def _mesh():
    return plsc.VectorSubcoreMesh(core_axis_name="c", subcore_axis_name="s")


_SC_PARAMS = pltpu.CompilerParams(use_tc_tiling_on_sc=False)


def _make_deg_kernel(C=80):
    @functools.partial(
        pl.kernel,
        mesh=_mesh(),
        compiler_params=_SC_PARAMS,
        out_type=jax.ShapeDtypeStruct((NC, N_PAD, DEG_W), F32),
        scratch_types=[
            pltpu.VMEM((C,), jnp.int32),
            pltpu.VMEM((C, DEG_W), F32),
            pltpu.VMEM_SHARED((N_PAD, DEG_W), F32),
        ],
    )
    def deg_kernel(dst_hbm, out_hbm, idx_v, buf_v, acc):
        c = lax.axis_index("c")
        s = lax.axis_index("s")
        wid = s * NC + c
        rpt = N_PAD // NS

        def zrow(i, _):
            buf_v[i, :] = jnp.zeros((DEG_W,), F32)
            return 0
        lax.fori_loop(0, C, zrow, 0)

        def zdma(i, _):
            pltpu.sync_copy(buf_v, acc.at[pl.ds(s * rpt + i * C, C)])
            return 0
        lax.fori_loop(0, rpt // C, zdma, 0)

        def orow(i, _):
            buf_v[i, :] = jnp.ones((DEG_W,), F32)
            return 0
        lax.fori_loop(0, C, orow, 0)
        plsc.subcore_barrier()

        def chunk(i, _):
            base = wid * EPW + i * C
            pltpu.sync_copy(dst_hbm.at[pl.ds(base, C)], idx_v)
            pltpu.sync_copy(buf_v, acc.at[idx_v], add=True)
            return 0
        lax.fori_loop(0, EPW // C, chunk, 0)
        plsc.subcore_barrier()

        pltpu.sync_copy(acc.at[pl.ds(s * rpt, rpt)],
                        out_hbm.at[c, pl.ds(s * rpt, rpt)])

    return deg_kernel


def _make_segsum_kernel(D, C=80):
    @functools.partial(
        pl.kernel,
        mesh=_mesh(),
        compiler_params=_SC_PARAMS,
        out_type=jax.ShapeDtypeStruct((NC, N_PAD, D), F32),
        scratch_types=[
            pltpu.VMEM((C,), jnp.int32),
            pltpu.VMEM((C,), jnp.int32),
            pltpu.VMEM((C, D), F32),
            pltpu.VMEM_SHARED((N_PAD, D), F32),
            pltpu.SemaphoreType.DMA,
        ],
    )
    def segsum_kernel(tab_hbm, src_hbm, dst_hbm, out_hbm,
                      si_v, di_v, rows_v, acc, sem):
        c = lax.axis_index("c")
        s = lax.axis_index("s")
        wid = s * NC + c
        rpt = N_PAD // NS

        def zrow(i, _):
            for j in range(D // 16):
                rows_v[i, pl.ds(j * 16, 16)] = jnp.zeros((16,), F32)
            return 0
        lax.fori_loop(0, C, zrow, 0)

        def zdma(i, _):
            pltpu.sync_copy(rows_v, acc.at[pl.ds(s * rpt + i * C, C)])
            return 0
        lax.fori_loop(0, rpt // C, zdma, 0)
        plsc.subcore_barrier()

        def chunk(i, _):
            base = wid * EPW + i * C
            pltpu.sync_copy(src_hbm.at[pl.ds(base, C)], si_v)
            pltpu.sync_copy(dst_hbm.at[pl.ds(base, C)], di_v)
            pltpu.async_copy(tab_hbm.at[si_v], rows_v, sem).wait()
            pltpu.sync_copy(rows_v, acc.at[di_v], add=True)
            return 0
        lax.fori_loop(0, EPW // C, chunk, 0)
        plsc.subcore_barrier()

        pltpu.sync_copy(acc.at[pl.ds(s * rpt, rpt)],
                        out_hbm.at[c, pl.ds(s * rpt, rpt)])

    return segsum_kernel


def _t1_body(degp, x, w1, wr, dinv_o, h1p_o, xwr_o):
    deg = degp[0, :, 0] + degp[1, :, 0] + 1.0
    di = lax.rsqrt(deg)
    dinv_o[...] = di[:, None]
    h1p_o[...] = di[:, None] * jnp.dot(x[...], w1[...],
                                       preferred_element_type=F32)
    xwr_o[...] = jnp.dot(x[...], wr[...], preferred_element_type=F32)


def _t2_body(s1, h1p, dinv, b1, w2, h2p_o):
    ssum = s1[0] + s1[1] + h1p[...]
    r = jnp.maximum(dinv[...] * ssum + b1[...], 0.0)
    h2p_o[...] = dinv[...] * jnp.dot(r, w2[...], preferred_element_type=F32)


def _t3_body(s2, h2p, xwr, dinv, b2, z_o, p_o):
    h = dinv[...] * (s2[0] + s2[1] + h2p[...]) + b2[...]
    z = h + xwr[...]
    nrm = jnp.sqrt(jnp.sum(z * z, axis=1, keepdims=True))
    z = z / jnp.maximum(nrm, 1e-12)
    m = jnp.max(z, axis=1, keepdims=True)
    e = jnp.exp(z - m)
    p_o[...] = e / jnp.sum(e, axis=1, keepdims=True)
    z_o[...] = z


def _row_spec(w):
    return pl.BlockSpec((R_BLK, w), lambda j: (j, 0))


def _full_spec(h, w):
    return pl.BlockSpec((h, w), lambda j: (0, 0))


def _pair_spec(w):
    return pl.BlockSpec((2, R_BLK, w), lambda j: (0, j, 0))


_GRID = (N_NODES // R_BLK,)


def _t1_call(degp, x, w1, wr):
    return pl.pallas_call(
        _t1_body,
        grid=_GRID,
        in_specs=[_pair_spec(DEG_W), _row_spec(128),
                  _full_spec(128, 128), _full_spec(128, 16)],
        out_specs=[_row_spec(1), _row_spec(128), _row_spec(16)],
        out_shape=[jax.ShapeDtypeStruct((N_NODES, 1), F32),
                   jax.ShapeDtypeStruct((N_NODES, 128), F32),
                   jax.ShapeDtypeStruct((N_NODES, 16), F32)],
    )(degp, x, w1, wr)


def _t2_call(s1, h1p, dinv, b1, w2):
    return pl.pallas_call(
        _t2_body,
        grid=_GRID,
        in_specs=[_pair_spec(128), _row_spec(128), _row_spec(1),
                  _full_spec(1, 128), _full_spec(128, 16)],
        out_specs=_row_spec(16),
        out_shape=jax.ShapeDtypeStruct((N_NODES, 16), F32),
    )(s1, h1p, dinv, b1, w2)


def _t3_call(s2, h2p, xwr, dinv, b2):
    return pl.pallas_call(
        _t3_body,
        grid=_GRID,
        in_specs=[_pair_spec(16), _row_spec(16), _row_spec(16),
                  _row_spec(1), _full_spec(1, 16)],
        out_specs=[_row_spec(16), _row_spec(16)],
        out_shape=[jax.ShapeDtypeStruct((N_NODES, 16), F32),
                   jax.ShapeDtypeStruct((N_NODES, 16), F32)],
    )(s2, h2p, xwr, dinv, b2)


_deg_call = _make_deg_kernel()
_seg128_call = _make_segsum_kernel(128)
_seg16_call = _make_segsum_kernel(16)


def kernel(x, edge_index, W1, b1, W2, b2, Wr):
    src = edge_index[0]
    dst = edge_index[1]
    degp = _deg_call(dst)
    dinv, h1p, xwr = _t1_call(degp[:, :N_NODES, :], x, W1, Wr)
    s1 = _seg128_call(h1p, src, dst)
    h2p = _t2_call(s1[:, :N_NODES, :], h1p, dinv,
                   b1.reshape(1, -1), W2)
    s2 = _seg16_call(h2p, src, dst)
    z, p = _t3_call(s2[:, :N_NODES, :], h2p, xwr, dinv,
                    b2.reshape(1, -1))
    return (z, p)


# speedup vs baseline: 28.8681x; 2.0778x over previous
import functools

import jax
import jax.numpy as jnp
from jax import lax
from jax.experimental import pallas as pl
from jax.experimental.pallas import tpu as pltpu
from jax.experimental.pallas import tpu_sc as plsc

N_NODES = 10000
N_PAD = 10240
E_EDGES = 320000
NC = 2
NS = 16
NW = NC * NS
EPW = E_EDGES // NW
DEG_W = 16
R_BLK = 1000
F32 = jnp.float32


def _mesh():
    return plsc.VectorSubcoreMesh(core_axis_name="c", subcore_axis_name="s")


_SC_PARAMS = pltpu.CompilerParams(use_tc_tiling_on_sc=False)


C_CHUNK = 80
NCH = EPW // C_CHUNK


def _make_deg_kernel():
    C = C_CHUNK

    @functools.partial(
        pl.kernel,
        mesh=_mesh(),
        compiler_params=_SC_PARAMS,
        out_type=jax.ShapeDtypeStruct((NC, N_PAD, DEG_W), F32),
        scratch_types=[
            pltpu.VMEM((NCH, C), jnp.int32),
            pltpu.VMEM((C, DEG_W), F32),
            pltpu.VMEM_SHARED((N_PAD, DEG_W), F32),
        ],
    )
    def deg_kernel(dst_hbm, out_hbm, di_v, buf_v, acc):
        c = lax.axis_index("c")
        s = lax.axis_index("s")
        wid = s * NC + c
        rpt = N_PAD // NS

        pltpu.sync_copy(dst_hbm.at[wid], di_v)

        def zrow(i, _):
            buf_v[i, :] = jnp.zeros((DEG_W,), F32)
            return 0
        lax.fori_loop(0, C, zrow, 0)

        def zdma(i, _):
            pltpu.sync_copy(buf_v, acc.at[pl.ds(s * rpt + i * C, C)])
            return 0
        lax.fori_loop(0, rpt // C, zdma, 0)

        def orow(i, _):
            buf_v[i, :] = jnp.ones((DEG_W,), F32)
            return 0
        lax.fori_loop(0, C, orow, 0)
        plsc.subcore_barrier()

        def chunk(i, _):
            pltpu.sync_copy(buf_v, acc.at[di_v.at[i]], add=True)
            return 0
        lax.fori_loop(0, NCH, chunk, 0)
        plsc.subcore_barrier()

        pltpu.sync_copy(acc.at[pl.ds(s * rpt, rpt)],
                        out_hbm.at[c, pl.ds(s * rpt, rpt)])

    return deg_kernel


def _make_segsum_kernel(D, NB):
    C = C_CHUNK

    @functools.partial(
        pl.kernel,
        mesh=_mesh(),
        compiler_params=_SC_PARAMS,
        out_type=jax.ShapeDtypeStruct((NC, N_PAD, D), F32),
        scratch_types=[
            pltpu.VMEM((NCH, C), jnp.int32),
            pltpu.VMEM((NCH, C), jnp.int32),
            pltpu.VMEM((NB, C, D), F32),
            pltpu.VMEM_SHARED((N_PAD, D), F32),
            pltpu.SemaphoreType.DMA((NB,)),
        ],
    )
    def segsum_kernel(tab_hbm, src_hbm, dst_hbm, out_hbm,
                      si_v, di_v, rows_v, acc, sems):
        c = lax.axis_index("c")
        s = lax.axis_index("s")
        wid = s * NC + c
        rpt = N_PAD // NS

        pltpu.sync_copy(src_hbm.at[wid], si_v)
        pltpu.sync_copy(dst_hbm.at[wid], di_v)

        def zrow(i, _):
            for j in range(D // 16):
                rows_v[0, i, pl.ds(j * 16, 16)] = jnp.zeros((16,), F32)
            return 0
        lax.fori_loop(0, C, zrow, 0)

        def zdma(i, _):
            pltpu.sync_copy(rows_v.at[0], acc.at[pl.ds(s * rpt + i * C, C)])
            return 0
        lax.fori_loop(0, rpt // C, zdma, 0)
        plsc.subcore_barrier()

        def do_chunks(base, nb):
            handles = [
                pltpu.async_copy(tab_hbm.at[si_v.at[base + b]],
                                 rows_v.at[b], sems.at[b])
                for b in range(nb)
            ]
            for b in range(nb):
                handles[b].wait()
                pltpu.sync_copy(rows_v.at[b], acc.at[di_v.at[base + b]],
                                add=True)

        def outer(o, _):
            do_chunks(o * NB, NB)
            return 0
        lax.fori_loop(0, NCH // NB, outer, 0)
        if NCH % NB:
            do_chunks(NCH - NCH % NB, NCH % NB)
        plsc.subcore_barrier()

        pltpu.sync_copy(acc.at[pl.ds(s * rpt, rpt)],
                        out_hbm.at[c, pl.ds(s * rpt, rpt)])

    return segsum_kernel


def _t1_body(degp, x, w1, wr, dinv_o, h1p_o, xwr_o):
    deg = degp[0, :, 0] + degp[1, :, 0] + 1.0
    di = lax.rsqrt(deg)
    dinv_o[...] = di[:, None]
    h1p_o[...] = di[:, None] * jnp.dot(x[...], w1[...],
                                       preferred_element_type=F32)
    xwr_o[...] = jnp.dot(x[...], wr[...], preferred_element_type=F32)


def _t2_body(s1, h1p, dinv, b1, w2, h2p_o):
    ssum = s1[0] + s1[1] + h1p[...]
    r = jnp.maximum(dinv[...] * ssum + b1[...], 0.0)
    h2p_o[...] = dinv[...] * jnp.dot(r, w2[...], preferred_element_type=F32)


def _t3_body(s2, h2p, xwr, dinv, b2, z_o, p_o):
    h = dinv[...] * (s2[0] + s2[1] + h2p[...]) + b2[...]
    z = h + xwr[...]
    nrm = jnp.sqrt(jnp.sum(z * z, axis=1, keepdims=True))
    z = z / jnp.maximum(nrm, 1e-12)
    m = jnp.max(z, axis=1, keepdims=True)
    e = jnp.exp(z - m)
    p_o[...] = e / jnp.sum(e, axis=1, keepdims=True)
    z_o[...] = z


def _row_spec(w):
    return pl.BlockSpec((R_BLK, w), lambda j: (j, 0))


def _full_spec(h, w):
    return pl.BlockSpec((h, w), lambda j: (0, 0))


def _pair_spec(w):
    return pl.BlockSpec((2, R_BLK, w), lambda j: (0, j, 0))


_GRID = (N_NODES // R_BLK,)


def _t1_call(degp, x, w1, wr):
    return pl.pallas_call(
        _t1_body,
        grid=_GRID,
        in_specs=[_pair_spec(DEG_W), _row_spec(128),
                  _full_spec(128, 128), _full_spec(128, 16)],
        out_specs=[_row_spec(1), _row_spec(128), _row_spec(16)],
        out_shape=[jax.ShapeDtypeStruct((N_NODES, 1), F32),
                   jax.ShapeDtypeStruct((N_NODES, 128), F32),
                   jax.ShapeDtypeStruct((N_NODES, 16), F32)],
    )(degp, x, w1, wr)


def _t2_call(s1, h1p, dinv, b1, w2):
    return pl.pallas_call(
        _t2_body,
        grid=_GRID,
        in_specs=[_pair_spec(128), _row_spec(128), _row_spec(1),
                  _full_spec(1, 128), _full_spec(128, 16)],
        out_specs=_row_spec(16),
        out_shape=jax.ShapeDtypeStruct((N_NODES, 16), F32),
    )(s1, h1p, dinv, b1, w2)


def _t3_call(s2, h2p, xwr, dinv, b2):
    return pl.pallas_call(
        _t3_body,
        grid=_GRID,
        in_specs=[_pair_spec(16), _row_spec(16), _row_spec(16),
                  _row_spec(1), _full_spec(1, 16)],
        out_specs=[_row_spec(16), _row_spec(16)],
        out_shape=[jax.ShapeDtypeStruct((N_NODES, 16), F32),
                   jax.ShapeDtypeStruct((N_NODES, 16), F32)],
    )(s2, h2p, xwr, dinv, b2)


_deg_call = _make_deg_kernel()
_seg128_call = _make_segsum_kernel(128, NB=2)
_seg16_call = _make_segsum_kernel(16, NB=5)


def kernel(x, edge_index, W1, b1, W2, b2, Wr):
    src = edge_index[0].reshape(NW, NCH, C_CHUNK)
    dst = edge_index[1].reshape(NW, NCH, C_CHUNK)
    degp = _deg_call(dst)
    dinv, h1p, xwr = _t1_call(degp[:, :N_NODES, :], x, W1, Wr)
    s1 = _seg128_call(h1p, src, dst)
    h2p = _t2_call(s1[:, :N_NODES, :], h1p, dinv,
                   b1.reshape(1, -1), W2)
    s2 = _seg16_call(h2p, src, dst)
    z, p = _t3_call(s2[:, :N_NODES, :], h2p, xwr, dinv,
                    b2.reshape(1, -1))
    return (z, p)


# speedup vs baseline: 38.4937x; 1.3334x over previous
import functools

import jax
import jax.numpy as jnp
from jax import lax
from jax.experimental import pallas as pl
from jax.experimental.pallas import tpu as pltpu
from jax.experimental.pallas import tpu_sc as plsc

N_NODES = 10000
N_PAD = 10240
E_EDGES = 320000
NC = 2
NS = 16
NW = NC * NS
EPW = E_EDGES // NW
DEG_W = 16
R_BLK = 1000
F32 = jnp.float32

C_CHUNK = 80
NCH = EPW // C_CHUNK


def _mesh():
    return plsc.VectorSubcoreMesh(core_axis_name="c", subcore_axis_name="s")


_SC_PARAMS = pltpu.CompilerParams(use_tc_tiling_on_sc=False)


def _make_deg_kernel():
    C = C_CHUNK

    @functools.partial(
        pl.kernel,
        mesh=_mesh(),
        compiler_params=_SC_PARAMS,
        out_type=jax.ShapeDtypeStruct((NC, N_PAD, DEG_W), F32),
        scratch_types=[
            pltpu.VMEM((NCH, C), jnp.int32),
            pltpu.VMEM((C, DEG_W), F32),
            pltpu.VMEM_SHARED((N_PAD, DEG_W), F32),
        ],
    )
    def deg_kernel(e4_hbm, out_hbm, di_v, buf_v, acc):
        c = lax.axis_index("c")
        s = lax.axis_index("s")
        wid = s * NC + c
        rpt = N_PAD // NS

        pltpu.sync_copy(e4_hbm.at[1, wid], di_v)

        def zrow(i, _):
            buf_v[i, :] = jnp.zeros((DEG_W,), F32)
            return 0
        lax.fori_loop(0, C, zrow, 0)

        def zdma(i, _):
            pltpu.sync_copy(buf_v, acc.at[pl.ds(s * rpt + i * C, C)])
            return 0
        lax.fori_loop(0, rpt // C, zdma, 0)

        def orow(i, _):
            buf_v[i, :] = jnp.ones((DEG_W,), F32)
            return 0
        lax.fori_loop(0, C, orow, 0)
        plsc.subcore_barrier()

        def chunk(i, _):
            pltpu.sync_copy(buf_v, acc.at[di_v.at[i]], add=True)
            return 0
        lax.fori_loop(0, NCH, chunk, 0)
        plsc.subcore_barrier()

        pltpu.sync_copy(acc.at[pl.ds(s * rpt, rpt)],
                        out_hbm.at[c, pl.ds(s * rpt, rpt)])

    return deg_kernel


def _make_segsum_kernel(D, NB):
    C = C_CHUNK

    @functools.partial(
        pl.kernel,
        mesh=_mesh(),
        compiler_params=_SC_PARAMS,
        out_type=jax.ShapeDtypeStruct((NC, N_PAD, D), F32),
        scratch_types=[
            pltpu.VMEM((NCH, C), jnp.int32),
            pltpu.VMEM((NCH, C), jnp.int32),
            pltpu.VMEM((NB, C, D), F32),
            pltpu.VMEM_SHARED((N_PAD, D), F32),
            pltpu.SemaphoreType.DMA((NB,)),
        ],
    )
    def segsum_kernel(tab_hbm, e4_hbm, out_hbm, si_v, di_v, rows_v, acc, sems):
        c = lax.axis_index("c")
        s = lax.axis_index("s")
        wid = s * NC + c
        rpt = N_PAD // NS

        pltpu.sync_copy(e4_hbm.at[0, wid], si_v)
        pltpu.sync_copy(e4_hbm.at[1, wid], di_v)

        def zrow(i, _):
            for j in range(D // 16):
                rows_v[0, i, pl.ds(j * 16, 16)] = jnp.zeros((16,), F32)
            return 0
        lax.fori_loop(0, C, zrow, 0)

        def zdma(i, _):
            pltpu.sync_copy(rows_v.at[0], acc.at[pl.ds(s * rpt + i * C, C)])
            return 0
        lax.fori_loop(0, rpt // C, zdma, 0)
        plsc.subcore_barrier()

        def gather(j, b):
            return pltpu.async_copy(tab_hbm.at[si_v.at[j]], rows_v.at[b],
                                    sems.at[b])

        def scatter(j, b):
            pltpu.sync_copy(rows_v.at[b], acc.at[di_v.at[j]], add=True)

        for b in range(NB):
            gather(b, b)
        main_iters = (NCH - NB) // NB

        def outer(o, _):
            for b in range(NB):
                j = o * NB + b
                pltpu.make_async_copy(tab_hbm.at[si_v.at[j]], rows_v.at[b],
                                      sems.at[b]).wait()
                scatter(j, b)
                gather(j + NB, b)
            return 0
        lax.fori_loop(0, main_iters, outer, 0)

        for t in range(NCH - main_iters * NB):
            j = main_iters * NB + t
            b = t % NB
            pltpu.make_async_copy(tab_hbm.at[si_v.at[j]], rows_v.at[b],
                                  sems.at[b]).wait()
            scatter(j, b)
            if j + NB < NCH:
                gather(j + NB, b)
        plsc.subcore_barrier()

        pltpu.sync_copy(acc.at[pl.ds(s * rpt, rpt)],
                        out_hbm.at[c, pl.ds(s * rpt, rpt)])

    return segsum_kernel


def _t1_body(degp, x, w1, wr, dinv_o, h1p_o, xwr_o):
    deg = degp[0, :, 0] + degp[1, :, 0] + 1.0
    di = lax.rsqrt(deg)
    dinv_o[...] = di[:, None]
    h1p_o[...] = di[:, None] * jnp.dot(x[...], w1[...],
                                       preferred_element_type=F32)
    xwr_o[...] = jnp.dot(x[...], wr[...], preferred_element_type=F32)


def _t2_body(s1, h1p, dinv, b1, w2, h2p_o):
    ssum = s1[0] + s1[1] + h1p[...]
    r = jnp.maximum(dinv[...] * ssum + b1[...], 0.0)
    h2p_o[...] = dinv[...] * jnp.dot(r, w2[...], preferred_element_type=F32)


def _t3_body(s2, h2p, xwr, dinv, b2, z_o, p_o):
    h = dinv[...] * (s2[0] + s2[1] + h2p[...]) + b2[...]
    z = h + xwr[...]
    nrm = jnp.sqrt(jnp.sum(z * z, axis=1, keepdims=True))
    z = z / jnp.maximum(nrm, 1e-12)
    m = jnp.max(z, axis=1, keepdims=True)
    e = jnp.exp(z - m)
    p_o[...] = e / jnp.sum(e, axis=1, keepdims=True)
    z_o[...] = z


def _row_spec(w):
    return pl.BlockSpec((R_BLK, w), lambda j: (j, 0))


def _full_spec(h, w):
    return pl.BlockSpec((h, w), lambda j: (0, 0))


def _pair_spec(w):
    return pl.BlockSpec((2, R_BLK, w), lambda j: (0, j, 0))


_GRID = (N_NODES // R_BLK,)


def _t1_call(degp, x, w1, wr):
    return pl.pallas_call(
        _t1_body,
        grid=_GRID,
        in_specs=[_pair_spec(DEG_W), _row_spec(128),
                  _full_spec(128, 128), _full_spec(128, 16)],
        out_specs=[_row_spec(1), _row_spec(128), _row_spec(16)],
        out_shape=[jax.ShapeDtypeStruct((N_NODES, 1), F32),
                   jax.ShapeDtypeStruct((N_NODES, 128), F32),
                   jax.ShapeDtypeStruct((N_NODES, 16), F32)],
    )(degp, x, w1, wr)


def _t2_call(s1, h1p, dinv, b1, w2):
    return pl.pallas_call(
        _t2_body,
        grid=_GRID,
        in_specs=[_pair_spec(128), _row_spec(128), _row_spec(1),
                  _full_spec(1, 128), _full_spec(128, 16)],
        out_specs=_row_spec(16),
        out_shape=jax.ShapeDtypeStruct((N_NODES, 16), F32),
    )(s1, h1p, dinv, b1, w2)


def _t3_call(s2, h2p, xwr, dinv, b2):
    return pl.pallas_call(
        _t3_body,
        grid=_GRID,
        in_specs=[_pair_spec(16), _row_spec(16), _row_spec(16),
                  _row_spec(1), _full_spec(1, 16)],
        out_specs=[_row_spec(16), _row_spec(16)],
        out_shape=[jax.ShapeDtypeStruct((N_NODES, 16), F32),
                   jax.ShapeDtypeStruct((N_NODES, 16), F32)],
    )(s2, h2p, xwr, dinv, b2)


_deg_call = _make_deg_kernel()
_seg128_call = _make_segsum_kernel(128, NB=2)
_seg16_call = _make_segsum_kernel(16, NB=5)


def kernel(x, edge_index, W1, b1, W2, b2, Wr):
    e4 = edge_index.reshape(2, NW, NCH, C_CHUNK)
    degp = _deg_call(e4)
    dinv, h1p, xwr = _t1_call(degp, x, W1, Wr)
    s1 = _seg128_call(h1p, e4)
    h2p = _t2_call(s1, h1p, dinv, b1.reshape(1, -1), W2)
    s2 = _seg16_call(h2p, e4)
    z, p = _t3_call(s2, h2p, xwr, dinv, b2.reshape(1, -1))
    return (z, p)


# speedup vs baseline: 38.5817x; 1.0023x over previous
import functools

import jax
import jax.numpy as jnp
from jax import lax
from jax.experimental import pallas as pl
from jax.experimental.pallas import tpu as pltpu
from jax.experimental.pallas import tpu_sc as plsc

N_NODES = 10000
N_PAD = 10240
E_EDGES = 320000
NC = 2
NS = 16
NW = NC * NS
EPW = E_EDGES // NW
DEG_W = 16
R_BLK = 1000
F32 = jnp.float32

C_CHUNK = 80
NCH = EPW // C_CHUNK


def _mesh():
    return plsc.VectorSubcoreMesh(core_axis_name="c", subcore_axis_name="s")


_SC_PARAMS = pltpu.CompilerParams(use_tc_tiling_on_sc=False)


def _make_deg_kernel():
    C = C_CHUNK

    @functools.partial(
        pl.kernel,
        mesh=_mesh(),
        compiler_params=_SC_PARAMS,
        out_type=jax.ShapeDtypeStruct((NC, N_PAD, DEG_W), F32),
        scratch_types=[
            pltpu.VMEM((NCH, C), jnp.int32),
            pltpu.VMEM((C, DEG_W), F32),
            pltpu.VMEM_SHARED((N_PAD, DEG_W), F32),
        ],
    )
    def deg_kernel(e4_hbm, out_hbm, di_v, buf_v, acc):
        c = lax.axis_index("c")
        s = lax.axis_index("s")
        wid = s * NC + c
        rpt = N_PAD // NS

        pltpu.sync_copy(e4_hbm.at[1, wid], di_v)

        def zrow(i, _):
            buf_v[i, :] = jnp.zeros((DEG_W,), F32)
            return 0
        lax.fori_loop(0, C, zrow, 0)

        def zdma(i, _):
            pltpu.sync_copy(buf_v, acc.at[pl.ds(s * rpt + i * C, C)])
            return 0
        lax.fori_loop(0, rpt // C, zdma, 0)

        def orow(i, _):
            buf_v[i, :] = jnp.ones((DEG_W,), F32)
            return 0
        lax.fori_loop(0, C, orow, 0)
        plsc.subcore_barrier()

        def chunk(i, _):
            pltpu.sync_copy(buf_v, acc.at[di_v.at[i]], add=True)
            return 0
        lax.fori_loop(0, NCH, chunk, 0)
        plsc.subcore_barrier()

        pltpu.sync_copy(acc.at[pl.ds(s * rpt, rpt)],
                        out_hbm.at[c, pl.ds(s * rpt, rpt)])

    return deg_kernel


def _make_segsum_kernel(D, NB, C=C_CHUNK):
    NCH = EPW // C

    @functools.partial(
        pl.kernel,
        mesh=_mesh(),
        compiler_params=_SC_PARAMS,
        out_type=jax.ShapeDtypeStruct((NC, N_PAD, D), F32),
        scratch_types=[
            pltpu.VMEM((NCH, C), jnp.int32),
            pltpu.VMEM((NCH, C), jnp.int32),
            pltpu.VMEM((NB, C, D), F32),
            pltpu.VMEM_SHARED((N_PAD, D), F32),
            pltpu.SemaphoreType.DMA((NB,)),
        ],
    )
    def segsum_kernel(tab_hbm, e4_hbm, out_hbm, si_v, di_v, rows_v, acc, sems):
        c = lax.axis_index("c")
        s = lax.axis_index("s")
        wid = s * NC + c
        rpt = N_PAD // NS

        pltpu.sync_copy(e4_hbm.at[0, wid], si_v)
        pltpu.sync_copy(e4_hbm.at[1, wid], di_v)

        def zrow(i, _):
            for j in range(D // 16):
                rows_v[0, i, pl.ds(j * 16, 16)] = jnp.zeros((16,), F32)
            return 0
        lax.fori_loop(0, C, zrow, 0)

        def zdma(i, _):
            pltpu.sync_copy(rows_v.at[0], acc.at[pl.ds(s * rpt + i * C, C)])
            return 0
        lax.fori_loop(0, rpt // C, zdma, 0)
        plsc.subcore_barrier()

        def gather(j, b):
            return pltpu.async_copy(tab_hbm.at[si_v.at[j]], rows_v.at[b],
                                    sems.at[b])

        def scatter(j, b):
            pltpu.sync_copy(rows_v.at[b], acc.at[di_v.at[j]], add=True)

        for b in range(NB):
            gather(b, b)
        main_iters = (NCH - NB) // NB

        def outer(o, _):
            for b in range(NB):
                j = o * NB + b
                pltpu.make_async_copy(tab_hbm.at[si_v.at[j]], rows_v.at[b],
                                      sems.at[b]).wait()
                scatter(j, b)
                gather(j + NB, b)
            return 0
        lax.fori_loop(0, main_iters, outer, 0)

        for t in range(NCH - main_iters * NB):
            j = main_iters * NB + t
            b = t % NB
            pltpu.make_async_copy(tab_hbm.at[si_v.at[j]], rows_v.at[b],
                                  sems.at[b]).wait()
            scatter(j, b)
            if j + NB < NCH:
                gather(j + NB, b)
        plsc.subcore_barrier()

        pltpu.sync_copy(acc.at[pl.ds(s * rpt, rpt)],
                        out_hbm.at[c, pl.ds(s * rpt, rpt)])

    return segsum_kernel


def _t1_body(degp, x, w1, wr, dinv_o, h1p_o, xwr_o):
    deg = degp[0, :, 0] + degp[1, :, 0] + 1.0
    di = lax.rsqrt(deg)
    dinv_o[...] = di[:, None]
    h1p_o[...] = di[:, None] * jnp.dot(x[...], w1[...],
                                       preferred_element_type=F32)
    xwr_o[...] = jnp.dot(x[...], wr[...], preferred_element_type=F32)


def _t2_body(s1, h1p, dinv, b1, w2, h2p_o):
    ssum = s1[0] + s1[1] + h1p[...]
    r = jnp.maximum(dinv[...] * ssum + b1[...], 0.0)
    h2p_o[...] = dinv[...] * jnp.dot(r, w2[...], preferred_element_type=F32)


def _t3_body(s2, h2p, xwr, dinv, b2, z_o, p_o):
    h = dinv[...] * (s2[0] + s2[1] + h2p[...]) + b2[...]
    z = h + xwr[...]
    nrm = jnp.sqrt(jnp.sum(z * z, axis=1, keepdims=True))
    z = z / jnp.maximum(nrm, 1e-12)
    m = jnp.max(z, axis=1, keepdims=True)
    e = jnp.exp(z - m)
    p_o[...] = e / jnp.sum(e, axis=1, keepdims=True)
    z_o[...] = z


def _row_spec(w):
    return pl.BlockSpec((R_BLK, w), lambda j: (j, 0))


def _full_spec(h, w):
    return pl.BlockSpec((h, w), lambda j: (0, 0))


def _pair_spec(w):
    return pl.BlockSpec((2, R_BLK, w), lambda j: (0, j, 0))


_GRID = (N_NODES // R_BLK,)


def _t1_call(degp, x, w1, wr):
    return pl.pallas_call(
        _t1_body,
        grid=_GRID,
        in_specs=[_pair_spec(DEG_W), _row_spec(128),
                  _full_spec(128, 128), _full_spec(128, 16)],
        out_specs=[_row_spec(1), _row_spec(128), _row_spec(16)],
        out_shape=[jax.ShapeDtypeStruct((N_NODES, 1), F32),
                   jax.ShapeDtypeStruct((N_NODES, 128), F32),
                   jax.ShapeDtypeStruct((N_NODES, 16), F32)],
    )(degp, x, w1, wr)


def _t2_call(s1, h1p, dinv, b1, w2):
    return pl.pallas_call(
        _t2_body,
        grid=_GRID,
        in_specs=[_pair_spec(128), _row_spec(128), _row_spec(1),
                  _full_spec(1, 128), _full_spec(128, 16)],
        out_specs=_row_spec(16),
        out_shape=jax.ShapeDtypeStruct((N_NODES, 16), F32),
    )(s1, h1p, dinv, b1, w2)


def _t3_call(s2, h2p, xwr, dinv, b2):
    return pl.pallas_call(
        _t3_body,
        grid=_GRID,
        in_specs=[_pair_spec(16), _row_spec(16), _row_spec(16),
                  _row_spec(1), _full_spec(1, 16)],
        out_specs=[_row_spec(16), _row_spec(16)],
        out_shape=[jax.ShapeDtypeStruct((N_NODES, 16), F32),
                   jax.ShapeDtypeStruct((N_NODES, 16), F32)],
    )(s2, h2p, xwr, dinv, b2)


_C128 = 40
_deg_call = _make_deg_kernel()
_seg128_call = _make_segsum_kernel(128, NB=3, C=_C128)
_seg16_call = _make_segsum_kernel(16, NB=5)


def kernel(x, edge_index, W1, b1, W2, b2, Wr):
    e4 = edge_index.reshape(2, NW, NCH, C_CHUNK)
    e4s = edge_index.reshape(2, NW, EPW // _C128, _C128)
    degp = _deg_call(e4)
    dinv, h1p, xwr = _t1_call(degp, x, W1, Wr)
    s1 = _seg128_call(h1p, e4s)
    h2p = _t2_call(s1, h1p, dinv, b1.reshape(1, -1), W2)
    s2 = _seg16_call(h2p, e4)
    z, p = _t3_call(s2, h2p, xwr, dinv, b2.reshape(1, -1))
    return (z, p)


# speedup vs baseline: 39.2311x; 1.0168x over previous
import functools

import jax
import jax.numpy as jnp
from jax import lax
from jax.experimental import pallas as pl
from jax.experimental.pallas import tpu as pltpu
from jax.experimental.pallas import tpu_sc as plsc

N_NODES = 10000
N_PAD = 10240
E_EDGES = 320000
NC = 2
NS = 16
NW = NC * NS
EPW = E_EDGES // NW
DEG_W = 8
R_BLK = 1000
F32 = jnp.float32

C_CHUNK = 80
NCH = EPW // C_CHUNK


def _mesh():
    return plsc.VectorSubcoreMesh(core_axis_name="c", subcore_axis_name="s")


_SC_PARAMS = pltpu.CompilerParams(use_tc_tiling_on_sc=False)


def _make_deg_kernel():
    C = C_CHUNK

    @functools.partial(
        pl.kernel,
        mesh=_mesh(),
        compiler_params=_SC_PARAMS,
        out_type=jax.ShapeDtypeStruct((NC, N_PAD, DEG_W), F32),
        scratch_types=[
            pltpu.VMEM((NCH, C), jnp.int32),
            pltpu.VMEM((2 * C, DEG_W), F32),
            pltpu.VMEM_SHARED((N_PAD, DEG_W), F32),
        ],
    )
    def deg_kernel(zo_hbm, e4_hbm, out_hbm, di_v, buf_v, acc):
        c = lax.axis_index("c")
        s = lax.axis_index("s")
        wid = s * NC + c
        rpt = N_PAD // NS

        pltpu.sync_copy(e4_hbm.at[1, wid], di_v)
        pltpu.sync_copy(zo_hbm, buf_v)

        def zdma(i, _):
            pltpu.sync_copy(buf_v.at[pl.ds(0, C)],
                            acc.at[pl.ds(s * rpt + i * C, C)])
            return 0
        lax.fori_loop(0, rpt // C, zdma, 0)
        plsc.subcore_barrier()

        def chunk(i, _):
            pltpu.sync_copy(buf_v.at[pl.ds(C, C)], acc.at[di_v.at[i]],
                            add=True)
            return 0
        lax.fori_loop(0, NCH, chunk, 0)
        plsc.subcore_barrier()

        pltpu.sync_copy(acc.at[pl.ds(s * rpt, rpt)],
                        out_hbm.at[c, pl.ds(s * rpt, rpt)])

    return deg_kernel


def _make_segsum_kernel(D, NB, C=C_CHUNK, tab_in_spmem=False):
    NCH = EPW // C
    NPT = N_NODES // NS

    @functools.partial(
        pl.kernel,
        mesh=_mesh(),
        compiler_params=_SC_PARAMS,
        out_type=jax.ShapeDtypeStruct((NC, N_PAD, D), F32),
        scratch_types=[
            pltpu.VMEM((NCH, C), jnp.int32),
            pltpu.VMEM((NCH, C), jnp.int32),
            pltpu.VMEM((NB, C, D), F32),
            pltpu.VMEM_SHARED((N_PAD, D), F32),
            pltpu.SemaphoreType.DMA((NB,)),
        ] + ([pltpu.VMEM_SHARED((N_NODES, D), F32)] if tab_in_spmem else []),
    )
    def segsum_kernel(tab_hbm, e4_hbm, out_hbm, si_v, di_v, rows_v, acc, sems,
                      *maybe_tab):
        c = lax.axis_index("c")
        s = lax.axis_index("s")
        wid = s * NC + c
        rpt = N_PAD // NS

        pltpu.sync_copy(e4_hbm.at[0, wid], si_v)
        pltpu.sync_copy(e4_hbm.at[1, wid], di_v)
        if tab_in_spmem:
            tab = maybe_tab[0]
            pltpu.sync_copy(tab_hbm.at[pl.ds(s * NPT, NPT)],
                            tab.at[pl.ds(s * NPT, NPT)])
        else:
            tab = tab_hbm

        def zrow(i, _):
            for j in range(D // 16):
                rows_v[0, i, pl.ds(j * 16, 16)] = jnp.zeros((16,), F32)
            return 0
        lax.fori_loop(0, C, zrow, 0)

        def zdma(i, _):
            pltpu.sync_copy(rows_v.at[0], acc.at[pl.ds(s * rpt + i * C, C)])
            return 0
        lax.fori_loop(0, rpt // C, zdma, 0)
        plsc.subcore_barrier()

        def gather(j, b):
            return pltpu.async_copy(tab.at[si_v.at[j]], rows_v.at[b],
                                    sems.at[b])

        def scatter(j, b):
            pltpu.sync_copy(rows_v.at[b], acc.at[di_v.at[j]], add=True)

        for b in range(NB):
            gather(b, b)
        main_iters = (NCH - NB) // NB

        def outer(o, _):
            for b in range(NB):
                j = o * NB + b
                pltpu.make_async_copy(tab_hbm.at[si_v.at[j]], rows_v.at[b],
                                      sems.at[b]).wait()
                scatter(j, b)
                gather(j + NB, b)
            return 0
        lax.fori_loop(0, main_iters, outer, 0)

        for t in range(NCH - main_iters * NB):
            j = main_iters * NB + t
            b = t % NB
            pltpu.make_async_copy(tab_hbm.at[si_v.at[j]], rows_v.at[b],
                                  sems.at[b]).wait()
            scatter(j, b)
            if j + NB < NCH:
                gather(j + NB, b)
        plsc.subcore_barrier()

        pltpu.sync_copy(acc.at[pl.ds(s * rpt, rpt)],
                        out_hbm.at[c, pl.ds(s * rpt, rpt)])

    return segsum_kernel


def _t1_body(degp, x, w1, wr, dinv_o, h1p_o, xwr_o):
    deg = degp[0, :, 0] + degp[1, :, 0] + 1.0
    di = lax.rsqrt(deg)
    dinv_o[...] = di[:, None]
    h1p_o[...] = di[:, None] * jnp.dot(x[...], w1[...],
                                       preferred_element_type=F32)
    xwr_o[...] = jnp.dot(x[...], wr[...], preferred_element_type=F32)


def _t2_body(s1, h1p, dinv, b1, w2, h2p_o):
    ssum = s1[0] + s1[1] + h1p[...]
    r = jnp.maximum(dinv[...] * ssum + b1[...], 0.0)
    h2p_o[...] = dinv[...] * jnp.dot(r, w2[...], preferred_element_type=F32)


def _t3_body(s2, h2p, xwr, dinv, b2, z_o, p_o):
    h = dinv[...] * (s2[0] + s2[1] + h2p[...]) + b2[...]
    z = h + xwr[...]
    nrm = jnp.sqrt(jnp.sum(z * z, axis=1, keepdims=True))
    z = z / jnp.maximum(nrm, 1e-12)
    m = jnp.max(z, axis=1, keepdims=True)
    e = jnp.exp(z - m)
    p_o[...] = e / jnp.sum(e, axis=1, keepdims=True)
    z_o[...] = z


def _row_spec(w):
    return pl.BlockSpec((R_BLK, w), lambda j: (j, 0))


def _full_spec(h, w):
    return pl.BlockSpec((h, w), lambda j: (0, 0))


def _pair_spec(w):
    return pl.BlockSpec((2, R_BLK, w), lambda j: (0, j, 0))


_GRID = (N_NODES // R_BLK,)


def _t1_call(degp, x, w1, wr):
    return pl.pallas_call(
        _t1_body,
        grid=_GRID,
        in_specs=[_pair_spec(DEG_W), _row_spec(128),
                  _full_spec(128, 128), _full_spec(128, 16)],
        out_specs=[_row_spec(1), _row_spec(128), _row_spec(16)],
        out_shape=[jax.ShapeDtypeStruct((N_NODES, 1), F32),
                   jax.ShapeDtypeStruct((N_NODES, 128), F32),
                   jax.ShapeDtypeStruct((N_NODES, 16), F32)],
    )(degp, x, w1, wr)


def _t2_call(s1, h1p, dinv, b1, w2):
    return pl.pallas_call(
        _t2_body,
        grid=_GRID,
        in_specs=[_pair_spec(128), _row_spec(128), _row_spec(1),
                  _full_spec(1, 128), _full_spec(128, 16)],
        out_specs=_row_spec(16),
        out_shape=jax.ShapeDtypeStruct((N_NODES, 16), F32),
    )(s1, h1p, dinv, b1, w2)


def _t3_call(s2, h2p, xwr, dinv, b2):
    return pl.pallas_call(
        _t3_body,
        grid=_GRID,
        in_specs=[_pair_spec(16), _row_spec(16), _row_spec(16),
                  _row_spec(1), _full_spec(1, 16)],
        out_specs=[_row_spec(16), _row_spec(16)],
        out_shape=[jax.ShapeDtypeStruct((N_NODES, 16), F32),
                   jax.ShapeDtypeStruct((N_NODES, 16), F32)],
    )(s2, h2p, xwr, dinv, b2)


_C128 = 40
_deg_call = _make_deg_kernel()
_seg128_call = _make_segsum_kernel(128, NB=3, C=_C128)
_seg16_call = _make_segsum_kernel(16, NB=5, tab_in_spmem=True)


def kernel(x, edge_index, W1, b1, W2, b2, Wr):
    e4 = edge_index.reshape(2, NW, NCH, C_CHUNK)
    e4s = edge_index.reshape(2, NW, EPW // _C128, _C128)
    zo = jnp.concatenate([jnp.zeros((C_CHUNK, DEG_W), F32),
                          jnp.ones((C_CHUNK, DEG_W), F32)])
    degp = _deg_call(zo, e4)
    dinv, h1p, xwr = _t1_call(degp, x, W1, Wr)
    s1 = _seg128_call(h1p, e4s)
    h2p = _t2_call(s1, h1p, dinv, b1.reshape(1, -1), W2)
    s2 = _seg16_call(h2p, e4)
    z, p = _t3_call(s2, h2p, xwr, dinv, b2.reshape(1, -1))
    return (z, p)
